# Initial kernel scaffold; baseline (speedup 1.0000x reference)
import jax
import jax.numpy as jnp
from jax.experimental import pallas as pl


def kernel(idx, scale_set, emb1, emb2, W1, b1, W2, b2):
    raise NotImplementedError("write your pallas kernel here")


# speedup vs baseline: 13.5901x; 13.5901x over previous
import functools

import jax
import jax.numpy as jnp
from jax.experimental import pallas as pl

_ALPHA = 3.0
_K = 64
_ROW_BLOCK = 256


def _nodevec_kernel(scale_ref, e1_ref, e2_ref, w1_ref, b1_ref, w2_ref, b2_ref,
                    nv1_ref, nv2_ref, nv1t_ref, nv2t_ref, *, fnum):
    v1 = e1_ref[...]
    v2 = e2_ref[...]
    for i in range(fnum):
        s = scale_ref[0:1, i:i + 1]
        dn = (((1,), (1,)), ((), ()))
        v1 = jnp.tanh(_ALPHA * (
            jax.lax.dot_general(v1 * s, w1_ref[i], dn,
                                preferred_element_type=jnp.float32)
            + b1_ref[i]))
        v2 = jnp.tanh(_ALPHA * (
            jax.lax.dot_general(v2 * s, w2_ref[i], dn,
                                preferred_element_type=jnp.float32)
            + b2_ref[i]))
        nv1_ref[i] = v1
        nv2_ref[i] = v2
        nv1t_ref[i] = v1.T
        nv2t_ref[i] = v2.T


def _adj_kernel(a1_ref, a2_ref, b1t_ref, b2t_ref, out_ref, *, n, k):
    br = a1_ref.shape[1]
    p = jnp.dot(a1_ref[0], b2t_ref[0], preferred_element_type=jnp.float32)
    q = jnp.dot(a2_ref[0], b1t_ref[0], preferred_element_type=jnp.float32)
    a = p - q
    v = jnp.where(a > 0, jnp.tanh(_ALPHA * a), 0.0)
    bits = jax.lax.bitcast_convert_type(v, jnp.int32)

    vchunk = v.reshape(br, n // 128, 128)
    cmax = jnp.max(vchunk, axis=2)
    rmax = jnp.max(cmax, axis=1, keepdims=True)
    cmin = jnp.min(cmax, axis=1, keepdims=True)
    cmin_bits = jax.lax.bitcast_convert_type(cmin, jnp.int32)
    cnt_min = jnp.sum((bits >= cmin_bits).astype(jnp.int32), axis=1,
                      keepdims=True)
    lo0 = jnp.where(cnt_min >= k, cmin_bits, 0)
    hi0 = jax.lax.bitcast_convert_type(rmax, jnp.int32)

    def cond(carry):
        lo, hi = carry
        return jnp.any(lo < hi)

    def body(carry):
        lo, hi = carry
        mid = lo + ((hi - lo + 1) >> 1)
        cnt = jnp.sum((bits >= mid).astype(jnp.int32), axis=1, keepdims=True)
        ge = cnt >= k
        return jnp.where(ge, mid, lo), jnp.where(ge, hi, mid - 1)

    t, _ = jax.lax.while_loop(cond, body, (lo0, hi0))

    gt = bits > t
    cnt_gt = jnp.sum(gt.astype(jnp.int32), axis=1, keepdims=True)
    m = (k - cnt_gt).astype(jnp.float32)
    eq = bits == t
    eqf = eq.astype(jnp.float32)

    nchunk = n // 128
    ri = jax.lax.broadcasted_iota(jnp.int32, (128, 128), 0)
    ci = jax.lax.broadcasted_iota(jnp.int32, (128, 128), 1)
    u128 = (ri < ci).astype(jnp.float32)
    ri2 = jax.lax.broadcasted_iota(jnp.int32, (nchunk, nchunk), 0)
    ci2 = jax.lax.broadcasted_iota(jnp.int32, (nchunk, nchunk), 1)
    uc = (ri2 < ci2).astype(jnp.float32)
    within = jnp.dot(eqf.reshape(br * nchunk, 128), u128,
                     preferred_element_type=jnp.float32)
    within = within.reshape(br, nchunk, 128)
    tot = jnp.sum(eqf.reshape(br, nchunk, 128), axis=2)
    off = jnp.dot(tot, uc, preferred_element_type=jnp.float32)
    rank = (off[:, :, None] + within).reshape(br, n)

    keep = gt | (eq & (rank < m))
    out_ref[0] = jnp.where(keep, v, 0.0)


@functools.partial(jax.jit, static_argnames=())
def kernel(idx, scale_set, emb1, emb2, W1, b1, W2, b2):
    n, dim = emb1.shape
    fnum = W1.shape[0]
    br = min(_ROW_BLOCK, n)
    nb = n // br

    nv1 = jnp.take(emb1, idx, axis=0)
    nv2 = jnp.take(emb2, idx, axis=0)

    nv1o, nv2o, nv1t, nv2t = pl.pallas_call(
        functools.partial(_nodevec_kernel, fnum=fnum),
        grid=(nb,),
        in_specs=[
            pl.BlockSpec((1, fnum), lambda r: (0, 0)),
            pl.BlockSpec((br, dim), lambda r: (r, 0)),
            pl.BlockSpec((br, dim), lambda r: (r, 0)),
            pl.BlockSpec((fnum, dim, dim), lambda r: (0, 0, 0)),
            pl.BlockSpec((fnum, 1, dim), lambda r: (0, 0, 0)),
            pl.BlockSpec((fnum, dim, dim), lambda r: (0, 0, 0)),
            pl.BlockSpec((fnum, 1, dim), lambda r: (0, 0, 0)),
        ],
        out_specs=[
            pl.BlockSpec((fnum, br, dim), lambda r: (0, r, 0)),
            pl.BlockSpec((fnum, br, dim), lambda r: (0, r, 0)),
            pl.BlockSpec((fnum, dim, br), lambda r: (0, 0, r)),
            pl.BlockSpec((fnum, dim, br), lambda r: (0, 0, r)),
        ],
        out_shape=[
            jax.ShapeDtypeStruct((fnum, n, dim), jnp.float32),
            jax.ShapeDtypeStruct((fnum, n, dim), jnp.float32),
            jax.ShapeDtypeStruct((fnum, dim, n), jnp.float32),
            jax.ShapeDtypeStruct((fnum, dim, n), jnp.float32),
        ],
    )(scale_set.reshape(1, fnum), nv1, nv2, W1, b1.reshape(fnum, 1, dim),
      W2, b2.reshape(fnum, 1, dim))

    adj = pl.pallas_call(
        functools.partial(_adj_kernel, n=n, k=_K),
        grid=(fnum, nb),
        in_specs=[
            pl.BlockSpec((1, br, dim), lambda i, r: (i, r, 0)),
            pl.BlockSpec((1, br, dim), lambda i, r: (i, r, 0)),
            pl.BlockSpec((1, dim, n), lambda i, r: (i, 0, 0)),
            pl.BlockSpec((1, dim, n), lambda i, r: (i, 0, 0)),
        ],
        out_specs=pl.BlockSpec((1, br, n), lambda i, r: (i, r, 0)),
        out_shape=jax.ShapeDtypeStruct((fnum, n, n), jnp.float32),
    )(nv1o, nv2o, nv1t, nv2t)

    return tuple(adj[i] for i in range(fnum))


# speedup vs baseline: 13.9209x; 1.0243x over previous
import functools

import jax
import jax.numpy as jnp
from jax.experimental import pallas as pl

_ALPHA = 3.0
_K = 64
_ROW_BLOCK = 256


def _nodevec_kernel(scale_ref, e1_ref, e2_ref, w1_ref, b1_ref, w2_ref, b2_ref,
                    nv1_ref, nv2_ref, nv1t_ref, nv2t_ref, *, fnum):
    v1 = e1_ref[...]
    v2 = e2_ref[...]
    for i in range(fnum):
        s = scale_ref[0:1, i:i + 1]
        dn = (((1,), (1,)), ((), ()))
        v1 = jnp.tanh(_ALPHA * (
            jax.lax.dot_general(v1 * s, w1_ref[i], dn,
                                preferred_element_type=jnp.float32)
            + b1_ref[i]))
        v2 = jnp.tanh(_ALPHA * (
            jax.lax.dot_general(v2 * s, w2_ref[i], dn,
                                preferred_element_type=jnp.float32)
            + b2_ref[i]))
        nv1_ref[i] = v1
        nv2_ref[i] = v2
        nv1t_ref[i] = v1.T
        nv2t_ref[i] = v2.T


def _adj_kernel(a1_ref, a2_ref, b1t_ref, b2t_ref, out_ref, *, n, k):
    br = a1_ref.shape[1]
    p = jnp.dot(a1_ref[0], b2t_ref[0], preferred_element_type=jnp.float32)
    q = jnp.dot(a2_ref[0], b1t_ref[0], preferred_element_type=jnp.float32)
    a = p - q
    v = jnp.where(a > 0, jnp.tanh(_ALPHA * a), 0.0)
    bits = jax.lax.bitcast_convert_type(v, jnp.int32)

    macc = v[:, 0:128]
    for c in range(1, n // 128):
        macc = jnp.maximum(macc, v[:, c * 128:(c + 1) * 128])
    rmax = jnp.max(macc, axis=1, keepdims=True)
    rmin = jnp.min(macc, axis=1, keepdims=True)
    lo0 = jax.lax.bitcast_convert_type(rmin, jnp.int32)
    hi0 = jax.lax.bitcast_convert_type(rmax, jnp.int32)

    def cond(carry):
        lo, hi = carry
        return jnp.any(lo < hi)

    def body(carry):
        lo, hi = carry
        mid = lo + ((hi - lo + 1) >> 1)
        cnt = jnp.sum((bits >= mid).astype(jnp.int32), axis=1, keepdims=True)
        ge = cnt >= k
        return jnp.where(ge, mid, lo), jnp.where(ge, hi, mid - 1)

    t, _ = jax.lax.while_loop(cond, body, (lo0, hi0))

    gt = bits > t
    cnt_gt = jnp.sum(gt.astype(jnp.int32), axis=1, keepdims=True)
    m = (k - cnt_gt).astype(jnp.float32)
    eq = bits == t
    eqf = eq.astype(jnp.float32)

    nchunk = n // 128
    ri = jax.lax.broadcasted_iota(jnp.int32, (128, 128), 0)
    ci = jax.lax.broadcasted_iota(jnp.int32, (128, 128), 1)
    u128 = (ri < ci).astype(jnp.float32)
    ri2 = jax.lax.broadcasted_iota(jnp.int32, (nchunk, nchunk), 0)
    ci2 = jax.lax.broadcasted_iota(jnp.int32, (nchunk, nchunk), 1)
    uc = (ri2 < ci2).astype(jnp.float32)
    within = jnp.dot(eqf.reshape(br * nchunk, 128), u128,
                     preferred_element_type=jnp.float32)
    within = within.reshape(br, nchunk, 128)
    tot = jnp.sum(eqf.reshape(br, nchunk, 128), axis=2)
    off = jnp.dot(tot, uc, preferred_element_type=jnp.float32)
    rank = (off[:, :, None] + within).reshape(br, n)

    keep = gt | (eq & (rank < m))
    out_ref[0] = jnp.where(keep, v, 0.0)


@functools.partial(jax.jit, static_argnames=())
def kernel(idx, scale_set, emb1, emb2, W1, b1, W2, b2):
    n, dim = emb1.shape
    fnum = W1.shape[0]
    br = min(_ROW_BLOCK, n)
    nb = n // br

    nv1 = jnp.take(emb1, idx, axis=0)
    nv2 = jnp.take(emb2, idx, axis=0)

    nv1o, nv2o, nv1t, nv2t = pl.pallas_call(
        functools.partial(_nodevec_kernel, fnum=fnum),
        grid=(nb,),
        in_specs=[
            pl.BlockSpec((1, fnum), lambda r: (0, 0)),
            pl.BlockSpec((br, dim), lambda r: (r, 0)),
            pl.BlockSpec((br, dim), lambda r: (r, 0)),
            pl.BlockSpec((fnum, dim, dim), lambda r: (0, 0, 0)),
            pl.BlockSpec((fnum, 1, dim), lambda r: (0, 0, 0)),
            pl.BlockSpec((fnum, dim, dim), lambda r: (0, 0, 0)),
            pl.BlockSpec((fnum, 1, dim), lambda r: (0, 0, 0)),
        ],
        out_specs=[
            pl.BlockSpec((fnum, br, dim), lambda r: (0, r, 0)),
            pl.BlockSpec((fnum, br, dim), lambda r: (0, r, 0)),
            pl.BlockSpec((fnum, dim, br), lambda r: (0, 0, r)),
            pl.BlockSpec((fnum, dim, br), lambda r: (0, 0, r)),
        ],
        out_shape=[
            jax.ShapeDtypeStruct((fnum, n, dim), jnp.float32),
            jax.ShapeDtypeStruct((fnum, n, dim), jnp.float32),
            jax.ShapeDtypeStruct((fnum, dim, n), jnp.float32),
            jax.ShapeDtypeStruct((fnum, dim, n), jnp.float32),
        ],
    )(scale_set.reshape(1, fnum), nv1, nv2, W1, b1.reshape(fnum, 1, dim),
      W2, b2.reshape(fnum, 1, dim))

    adj = pl.pallas_call(
        functools.partial(_adj_kernel, n=n, k=_K),
        grid=(fnum, nb),
        in_specs=[
            pl.BlockSpec((1, br, dim), lambda i, r: (i, r, 0)),
            pl.BlockSpec((1, br, dim), lambda i, r: (i, r, 0)),
            pl.BlockSpec((1, dim, n), lambda i, r: (i, 0, 0)),
            pl.BlockSpec((1, dim, n), lambda i, r: (i, 0, 0)),
        ],
        out_specs=pl.BlockSpec((1, br, n), lambda i, r: (i, r, 0)),
        out_shape=jax.ShapeDtypeStruct((fnum, n, n), jnp.float32),
    )(nv1o, nv2o, nv1t, nv2t)

    return tuple(adj[i] for i in range(fnum))
